# baseline (speedup 1.0000x reference)
import functools

import jax
import jax.numpy as jnp
from jax import lax
from jax.experimental import pallas as pl
from jax.experimental.pallas import tpu as pltpu
from jax.experimental.pallas import tpu_sc as plsc

_HIDDEN = 768
_EXPERTS = 64
_BT = 4096
_TOKENS = 32768


def _logits_body(h_ref, w1_ref, b1_ref, w2t_ref, b2_ref, out_ref):
    a1 = jnp.tanh(
        jnp.dot(h_ref[...], w1_ref[...], preferred_element_type=jnp.float32)
        + b1_ref[...]
    )
    logits = (
        lax.dot_general(
            a1, w2t_ref[...], (((1,), (1,)), ((), ())),
            preferred_element_type=jnp.float32,
        )
        + b2_ref[...]
    )
    out_ref[...] = logits.T


@jax.jit
def _logits_t(h, W1, b1, W2, b2):
    tokens = h.shape[0]
    grid = (tokens // _BT,)
    return pl.pallas_call(
        _logits_body,
        grid=grid,
        in_specs=[
            pl.BlockSpec((_BT, _HIDDEN), lambda i: (i, 0)),
            pl.BlockSpec((_HIDDEN, _HIDDEN), lambda i: (0, 0)),
            pl.BlockSpec((_HIDDEN,), lambda i: (0,)),
            pl.BlockSpec((_EXPERTS, _HIDDEN), lambda i: (0, 0)),
            pl.BlockSpec((1, _EXPERTS), lambda i: (0, 0)),
        ],
        out_specs=pl.BlockSpec((_EXPERTS, _BT), lambda i: (0, i)),
        out_shape=jax.ShapeDtypeStruct((_EXPERTS, tokens), jnp.float32),
    )(h, W1, b1, W2.T, b2.reshape(1, _EXPERTS))


_NC = 2
_NS = 16
_NW = _NC * _NS
_TPW = _TOKENS // _NW
_CHT = 256
_L = 16


def _sc_body(lt_hbm, out_hbm, chunk_v, out_v):
    wid = lax.axis_index("s") * _NC + lax.axis_index("c")
    base = wid * _TPW

    def chunk_body(c, carry):
        off = base + c * _CHT
        pltpu.sync_copy(lt_hbm.at[:, pl.ds(off, _CHT)], chunk_v)

        def group_body(g, carry2):
            t0 = g * _L
            v = [chunk_v[e, pl.ds(t0, _L)] for e in range(_EXPERTS)]
            m1 = v[0]
            for x in v[1:]:
                m1 = jnp.maximum(m1, x)
            neg = jnp.full((_L,), -jnp.inf, jnp.float32)
            m2 = jnp.where(v[0] == m1, neg, v[0])
            for x in v[1:]:
                m2 = jnp.maximum(m2, jnp.where(x == m1, neg, x))
            t = jnp.exp(m2 - m1)
            p2 = t / (1.0 + t)
            p1 = 1.0 - p2
            zero = jnp.zeros((_L,), jnp.float32)
            for e, x in enumerate(v):
                out_v[e, pl.ds(t0, _L)] = jnp.where(
                    x == m1, p1, jnp.where(x >= m2, p2, zero)
                )
            return carry2

        lax.fori_loop(0, _CHT // _L, group_body, 0)
        pltpu.sync_copy(out_v, out_hbm.at[:, pl.ds(off, _CHT)])
        return carry

    lax.fori_loop(0, _TPW // _CHT, chunk_body, 0)


_sc_top2 = functools.partial(
    pl.kernel,
    mesh=plsc.VectorSubcoreMesh(core_axis_name="c", subcore_axis_name="s"),
    out_type=jax.ShapeDtypeStruct((_EXPERTS, _TOKENS), jnp.float32),
    scratch_types=[
        pltpu.VMEM((_EXPERTS, _CHT), jnp.float32),
        pltpu.VMEM((_EXPERTS, _CHT), jnp.float32),
    ],
)(_sc_body)


def kernel(h, W1, b1, W2, b2, epoch, top_k):
    lt = _logits_t(h, W1, b1, W2, b2)
    return _sc_top2(lt).T


# speedup vs baseline: 1.7521x; 1.7521x over previous
import jax
import jax.numpy as jnp
from jax import lax
from jax.experimental import pallas as pl

_HIDDEN = 768
_EXPERTS = 64
_BT = 4096


def _gating_body(flag_ref, h_ref, w1_ref, b1_ref, w2t_ref, b2_ref, out_ref):
    a1 = jnp.tanh(
        jnp.dot(h_ref[...], w1_ref[...], preferred_element_type=jnp.float32)
        + b1_ref[...]
    )
    lt = (
        lax.dot_general(
            a1, w2t_ref[...], (((1,), (1,)), ((), ())),
            preferred_element_type=jnp.float32,
        )
        + b2_ref[...]
    ).T

    m1 = jnp.max(lt, axis=0, keepdims=True)
    is_max = lt == m1
    m2 = jnp.max(jnp.where(is_max, -jnp.inf, lt), axis=0, keepdims=True)

    use_dense = flag_ref[0, 0] != 0

    @pl.when(jnp.logical_not(use_dense))
    def _sparse():
        t = jnp.exp(m2 - m1)
        p2 = t / (1.0 + t)
        p1 = 1.0 - p2
        out_ref[...] = jnp.where(is_max, p1, jnp.where(lt >= m2, p2, 0.0))

    @pl.when(use_dense)
    def _dense():
        e = jnp.exp(lt - m1)
        out_ref[...] = e / jnp.sum(e, axis=0, keepdims=True)


@jax.jit
def _gating(h, W1, b1, W2, b2, flag):
    tokens = h.shape[0]
    grid = (tokens // _BT,)
    out_t = pl.pallas_call(
        _gating_body,
        grid=grid,
        in_specs=[
            pl.BlockSpec((1, 1), lambda i: (0, 0)),
            pl.BlockSpec((_BT, _HIDDEN), lambda i: (i, 0)),
            pl.BlockSpec((_HIDDEN, _HIDDEN), lambda i: (0, 0)),
            pl.BlockSpec((_HIDDEN,), lambda i: (0,)),
            pl.BlockSpec((_EXPERTS, _HIDDEN), lambda i: (0, 0)),
            pl.BlockSpec((1, _EXPERTS), lambda i: (0, 0)),
        ],
        out_specs=pl.BlockSpec((_EXPERTS, _BT), lambda i: (0, i)),
        out_shape=jax.ShapeDtypeStruct((_EXPERTS, tokens), jnp.float32),
    )(flag, h, W1, b1, W2.T, b2.reshape(1, _EXPERTS))
    return out_t.T


def kernel(h, W1, b1, W2, b2, epoch, top_k):
    warmup_epochs = 0
    if epoch is None or top_k is None:
        flag = jnp.ones((1, 1), jnp.float32)
    else:
        use_dense = (epoch < warmup_epochs) | (top_k <= 0)
        flag = jnp.asarray(use_dense, jnp.float32).reshape(1, 1)
    return _gating(h, W1, b1, W2, b2, flag)
